# Initial kernel scaffold; baseline (speedup 1.0000x reference)
import jax
import jax.numpy as jnp
from jax.experimental import pallas as pl


def kernel(x, edge_index, edge_attr, weight, bias):
    raise NotImplementedError("write your pallas kernel here")


# speedup vs baseline: 4.0418x; 4.0418x over previous
import functools

import jax
import jax.numpy as jnp
from jax import lax
from jax.experimental import pallas as pl
from jax.experimental.pallas import tpu as pltpu
from jax.experimental.pallas import tpu_sc as plsc

NC = 2
NS = 16
L = 16

N = 10000
E = 320000
D = 128
DH = D // NC

EPT = E // NS
B = 80
CH = EPT // B
NPT = 624
TAIL = N - NS * NPT
ZR = 104

_mesh = plsc.VectorSubcoreMesh(core_axis_name="c", subcore_axis_name="s")


@functools.partial(
    pl.kernel,
    out_type=(
        jax.ShapeDtypeStruct((NC, N, DH), jnp.float32),
        jax.ShapeDtypeStruct((NC, N, L), jnp.float32),
    ),
    mesh=_mesh,
    compiler_params=pltpu.CompilerParams(use_tc_tiling_on_sc=False),
    scratch_types=(
        pltpu.VMEM((B,), jnp.int32),
        pltpu.VMEM((B,), jnp.int32),
        pltpu.VMEM((B,), jnp.float32),
        pltpu.VMEM((B, DH), jnp.float32),
        pltpu.VMEM((B, L), jnp.float32),
        pltpu.VMEM((ZR, DH), jnp.float32),
        pltpu.VMEM((ZR, L), jnp.float32),
        pltpu.VMEM_SHARED((N, DH), jnp.float32),
        pltpu.VMEM_SHARED((N, L), jnp.float32),
        pltpu.SemaphoreType.DMA,
    ),
)
def _sc_edge_scatter(y_hbm, row_hbm, col_hbm, ea_hbm, acc_out, sc_out,
                     row_v, col_v, ea_v, rows_v, scal_v, zrow_v, zsc_v,
                     acc_sh, sc_sh, sem):
    cid = lax.axis_index("c")
    sid = lax.axis_index("s")

    def _zero_fill(i, c):
        zero = jnp.zeros((L,), jnp.float32)
        for j in range(DH // L):
            zrow_v[i, pl.ds(j * L, L)] = zero
        zsc_v[i, :] = zero
        return c

    lax.fori_loop(0, ZR, _zero_fill, 0)

    nbase = pl.multiple_of(sid * NPT, 8)
    for k in range(NPT // ZR):
        pltpu.sync_copy(zrow_v, acc_sh.at[pl.ds(nbase + k * ZR, ZR)])
        pltpu.sync_copy(zsc_v, sc_sh.at[pl.ds(nbase + k * ZR, ZR)])

    @pl.when(sid == NS - 1)
    def _zero_tail():
        pltpu.sync_copy(zrow_v.at[pl.ds(0, TAIL)], acc_sh.at[pl.ds(NS * NPT, TAIL)])
        pltpu.sync_copy(zsc_v.at[pl.ds(0, TAIL)], sc_sh.at[pl.ds(NS * NPT, TAIL)])

    plsc.subcore_barrier()

    ebase = sid * EPT

    def _chunk(k, c):
        off = ebase + k * B
        pltpu.sync_copy(row_hbm.at[pl.ds(off, B)], row_v)
        pltpu.sync_copy(col_hbm.at[pl.ds(off, B)], col_v)
        pltpu.sync_copy(ea_hbm.at[pl.ds(off, B)], ea_v)

        def _rebase(g, c2):
            gbase = pl.multiple_of(g * L, L)
            col_v[pl.ds(gbase, L)] = col_v[pl.ds(gbase, L)] + cid * N
            return c2

        lax.fori_loop(0, B // L, _rebase, 0)
        pltpu.async_copy(y_hbm.at[col_v], rows_v, sem).wait()

        def _group(g, c2):
            gbase = pl.multiple_of(g * L, L)
            ea16 = ea_v[pl.ds(gbase, L)]
            for l in range(L):
                e = gbase + l
                ea_bc = jnp.full((L,), ea16[l], jnp.float32)
                for j in range(DH // L):
                    rows_v[e, pl.ds(j * L, L)] = rows_v[e, pl.ds(j * L, L)] * ea_bc
            return c2

        lax.fori_loop(0, B // L, _group, 0)
        pltpu.sync_copy(rows_v, acc_sh.at[row_v], add=True)

        @pl.when(k % NC == cid)
        def _scal():
            def _sgroup(g, c3):
                gbase = pl.multiple_of(g * L, L)
                ea16 = ea_v[pl.ds(gbase, L)]
                iota = lax.iota(jnp.int32, L)
                for l in range(L):
                    ea_bc = jnp.full((L,), ea16[l], jnp.float32)
                    scal_v[gbase + l, :] = jnp.where(
                        iota == 0, ea_bc, jnp.where(iota == 1, 1.0, 0.0))
                return c3

            lax.fori_loop(0, B // L, _sgroup, 0)
            pltpu.sync_copy(scal_v, sc_sh.at[row_v], add=True)

        return c

    lax.fori_loop(0, CH, _chunk, 0)
    plsc.subcore_barrier()

    pltpu.sync_copy(acc_sh.at[pl.ds(nbase, NPT)], acc_out.at[cid, pl.ds(nbase, NPT)])
    pltpu.sync_copy(sc_sh.at[pl.ds(nbase, NPT)], sc_out.at[cid, pl.ds(nbase, NPT)])

    @pl.when(sid == NS - 1)
    def _write_tail():
        pltpu.sync_copy(acc_sh.at[pl.ds(NS * NPT, TAIL)],
                        acc_out.at[cid, pl.ds(NS * NPT, TAIL)])
        pltpu.sync_copy(sc_sh.at[pl.ds(NS * NPT, TAIL)],
                        sc_out.at[cid, pl.ds(NS * NPT, TAIL)])


def _mm_body(x_ref, w_ref, o_ref):
    o_ref[0] = jnp.dot(x_ref[...], w_ref[0], preferred_element_type=jnp.float32)


def _matmul_halves(x, w2h):
    g = 10
    bn = N // g
    return pl.pallas_call(
        _mm_body,
        grid=(NC, g),
        in_specs=[pl.BlockSpec((bn, D), lambda c, i: (i, 0)),
                  pl.BlockSpec((1, D, DH), lambda c, i: (c, 0, 0))],
        out_specs=pl.BlockSpec((1, bn, DH), lambda c, i: (c, i, 0)),
        out_shape=jax.ShapeDtypeStruct((NC, N, DH), jnp.float32),
    )(x, w2h)


def _fin_body(x_ref, w_ref, acc_ref, sc_ref, b_ref, o_ref):
    z1 = jnp.dot(x_ref[...], w_ref[...], preferred_element_type=jnp.float32)
    scat = jnp.concatenate([acc_ref[0], acc_ref[1]], axis=-1)
    srow = sc_ref[0] + sc_ref[1]
    sea = srow[:, 0:1]
    cnt = jnp.maximum(srow[:, 1:2], 1.0)
    o_ref[...] = (sea * z1 + scat) / cnt + b_ref[...]


def _finale(x, w1, acc, sc, bias):
    g = 10
    bn = N // g
    return pl.pallas_call(
        _fin_body,
        grid=(g,),
        in_specs=[
            pl.BlockSpec((bn, D), lambda i: (i, 0)),
            pl.BlockSpec((D, D), lambda i: (0, 0)),
            pl.BlockSpec((NC, bn, DH), lambda i: (0, i, 0)),
            pl.BlockSpec((NC, bn, L), lambda i: (0, i, 0)),
            pl.BlockSpec((1, D), lambda i: (0, 0)),
        ],
        out_specs=pl.BlockSpec((bn, D), lambda i: (i, 0)),
        out_shape=jax.ShapeDtypeStruct((N, D), jnp.float32),
    )(x, w1, acc, sc, bias.reshape(1, D))


def kernel(x, edge_index, edge_attr, weight, bias):
    row = edge_index[0]
    col = edge_index[1]
    w1 = weight[:D]
    w2 = weight[D:]
    w2h = jnp.stack([w2[:, :DH], w2[:, DH:]])
    y = _matmul_halves(x, w2h).reshape(NC * N, DH)
    acc, sc = _sc_edge_scatter(y, row, col, edge_attr)
    return _finale(x, w1, acc, sc, bias)


# speedup vs baseline: 6.5368x; 1.6173x over previous
import functools

import jax
import jax.numpy as jnp
from jax import lax
from jax.experimental import pallas as pl
from jax.experimental.pallas import tpu as pltpu
from jax.experimental.pallas import tpu_sc as plsc

NC = 2
NS = 16
L = 16

N = 10000
E = 320000
D = 128
DH = D // NC

EPT = E // NS
B = 80
NB = EPT // B
NPT = 624
TAIL = N - NS * NPT
ZR = 104

_mesh = plsc.VectorSubcoreMesh(core_axis_name="c", subcore_axis_name="s")


@functools.partial(
    pl.kernel,
    out_type=(
        jax.ShapeDtypeStruct((NC, N, DH), jnp.float32),
        jax.ShapeDtypeStruct((NC, N, L), jnp.float32),
    ),
    mesh=_mesh,
    compiler_params=pltpu.CompilerParams(use_tc_tiling_on_sc=False),
    scratch_types=(
        pltpu.VMEM((NB, B), jnp.int32),
        pltpu.VMEM((NB, B), jnp.int32),
        pltpu.VMEM((NB, B), jnp.float32),
        pltpu.VMEM((2, B, DH), jnp.float32),
        pltpu.VMEM((B, L), jnp.float32),
        pltpu.VMEM((ZR, DH), jnp.float32),
        pltpu.VMEM((ZR, L), jnp.float32),
        pltpu.VMEM_SHARED((N, DH), jnp.float32),
        pltpu.VMEM_SHARED((N, L), jnp.float32),
        pltpu.SemaphoreType.DMA,
        pltpu.SemaphoreType.DMA,
        pltpu.SemaphoreType.DMA,
        pltpu.SemaphoreType.DMA,
        pltpu.SemaphoreType.DMA,
        pltpu.SemaphoreType.DMA,
    ),
)
def _sc_edge_scatter(y_hbm, row_hbm, col_hbm, ea_hbm, acc_out, sc_out,
                     row_v, col_v, ea_v, rows_v, scal_v, zrow_v, zsc_v,
                     acc_sh, sc_sh, g0, g1, s0, s1, ss, pre):
    cid = lax.axis_index("c")
    sid = lax.axis_index("s")
    gsem = (g0, g1)
    ssem = (s0, s1)

    cbase = pl.multiple_of(sid * NB, 2)
    d_row = pltpu.async_copy(row_hbm.at[pl.ds(cbase, NB)], row_v, pre)
    d_col = pltpu.async_copy(col_hbm.at[pl.ds(cbase, NB)], col_v, pre)
    d_ea = pltpu.async_copy(ea_hbm.at[pl.ds(cbase, NB)], ea_v, pre)

    def _zero_fill(i, c):
        zero = jnp.zeros((L,), jnp.float32)
        for j in range(DH // L):
            zrow_v[i, pl.ds(j * L, L)] = zero
        zsc_v[i, :] = zero
        return c

    lax.fori_loop(0, ZR, _zero_fill, 0)

    nbase = pl.multiple_of(sid * NPT, 8)
    for k in range(NPT // ZR):
        pltpu.sync_copy(zrow_v, acc_sh.at[pl.ds(nbase + k * ZR, ZR)])
        pltpu.sync_copy(zsc_v, sc_sh.at[pl.ds(nbase + k * ZR, ZR)])

    @pl.when(sid == NS - 1)
    def _zero_tail():
        pltpu.sync_copy(zrow_v.at[pl.ds(0, TAIL)], acc_sh.at[pl.ds(NS * NPT, TAIL)])
        pltpu.sync_copy(zsc_v.at[pl.ds(0, TAIL)], sc_sh.at[pl.ds(NS * NPT, TAIL)])

    d_row.wait()
    d_col.wait()
    d_ea.wait()

    def _rebase(g, c):
        gbase = pl.multiple_of(g * L, L)
        m = g // (B // L)
        o = (g % (B // L)) * L
        col_v[m, pl.ds(o, L)] = col_v[m, pl.ds(o, L)] + cid * N
        return c

    lax.fori_loop(0, NB * (B // L), _rebase, 0)
    plsc.subcore_barrier()

    pltpu.async_copy(y_hbm.at[col_v.at[0]], rows_v.at[0], g0)

    def _outer(m, c):
        for b in range(2):
            k = 2 * m + b
            nb = 1 - b

            @pl.when(k >= 1)
            def _drain_prev_scatter():
                pltpu.make_async_copy(
                    rows_v.at[nb], acc_sh.at[row_v.at[k - 1]], ssem[nb]).wait()

            @pl.when(k + 1 < NB)
            def _start_next_gather():
                pltpu.async_copy(y_hbm.at[col_v.at[k + 1]], rows_v.at[nb], gsem[nb])

            pltpu.make_async_copy(
                y_hbm.at[col_v.at[k]], rows_v.at[b], gsem[b]).wait()

            def _group(g, c2):
                gbase = pl.multiple_of(g * L, L)
                ea16 = ea_v[k, pl.ds(gbase, L)]
                for l in range(L):
                    e = gbase + l
                    ea_bc = jnp.full((L,), ea16[l], jnp.float32)
                    for j in range(DH // L):
                        rows_v[b, e, pl.ds(j * L, L)] = (
                            rows_v[b, e, pl.ds(j * L, L)] * ea_bc)
                return c2

            lax.fori_loop(0, B // L, _group, 0)
            pltpu.async_copy(rows_v.at[b], acc_sh.at[row_v.at[k]], ssem[b],
                             add=True)

            @pl.when(k % NC == cid)
            def _scal():
                @pl.when(k >= 2)
                def _drain_prev():
                    pltpu.make_async_copy(
                        scal_v, sc_sh.at[row_v.at[k]], ss).wait()

                def _sgroup(g, c3):
                    gbase = pl.multiple_of(g * L, L)
                    ea16 = ea_v[k, pl.ds(gbase, L)]
                    iota = lax.iota(jnp.int32, L)
                    for l in range(L):
                        ea_bc = jnp.full((L,), ea16[l], jnp.float32)
                        scal_v[gbase + l, :] = jnp.where(
                            iota == 0, ea_bc, jnp.where(iota == 1, 1.0, 0.0))
                    return c3

                lax.fori_loop(0, B // L, _sgroup, 0)
                pltpu.async_copy(scal_v, sc_sh.at[row_v.at[k]], ss, add=True)

        return c

    lax.fori_loop(0, NB // 2, _outer, 0)

    pltpu.make_async_copy(rows_v.at[1], acc_sh.at[row_v.at[NB - 1]], s1).wait()
    pltpu.make_async_copy(scal_v, sc_sh.at[row_v.at[NB - 1]], ss).wait()
    plsc.subcore_barrier()

    pltpu.sync_copy(acc_sh.at[pl.ds(nbase, NPT)], acc_out.at[cid, pl.ds(nbase, NPT)])
    pltpu.sync_copy(sc_sh.at[pl.ds(nbase, NPT)], sc_out.at[cid, pl.ds(nbase, NPT)])

    @pl.when(sid == NS - 1)
    def _write_tail():
        pltpu.sync_copy(acc_sh.at[pl.ds(NS * NPT, TAIL)],
                        acc_out.at[cid, pl.ds(NS * NPT, TAIL)])
        pltpu.sync_copy(sc_sh.at[pl.ds(NS * NPT, TAIL)],
                        sc_out.at[cid, pl.ds(NS * NPT, TAIL)])


def _mm_body(x_ref, w_ref, o_ref):
    o_ref[0] = jnp.dot(x_ref[...], w_ref[0], preferred_element_type=jnp.float32)


def _matmul_halves(x, w2h):
    g = 10
    bn = N // g
    return pl.pallas_call(
        _mm_body,
        grid=(NC, g),
        in_specs=[pl.BlockSpec((bn, D), lambda c, i: (i, 0)),
                  pl.BlockSpec((1, D, DH), lambda c, i: (c, 0, 0))],
        out_specs=pl.BlockSpec((1, bn, DH), lambda c, i: (c, i, 0)),
        out_shape=jax.ShapeDtypeStruct((NC, N, DH), jnp.float32),
    )(x, w2h)


def _fin_body(x_ref, w_ref, acc_ref, sc_ref, b_ref, o_ref):
    z1 = jnp.dot(x_ref[...], w_ref[...], preferred_element_type=jnp.float32)
    scat = jnp.concatenate([acc_ref[0], acc_ref[1]], axis=-1)
    srow = sc_ref[0] + sc_ref[1]
    sea = srow[:, 0:1]
    cnt = jnp.maximum(srow[:, 1:2], 1.0)
    o_ref[...] = (sea * z1 + scat) / cnt + b_ref[...]


def _finale(x, w1, acc, sc, bias):
    g = 10
    bn = N // g
    return pl.pallas_call(
        _fin_body,
        grid=(g,),
        in_specs=[
            pl.BlockSpec((bn, D), lambda i: (i, 0)),
            pl.BlockSpec((D, D), lambda i: (0, 0)),
            pl.BlockSpec((NC, bn, DH), lambda i: (0, i, 0)),
            pl.BlockSpec((NC, bn, L), lambda i: (0, i, 0)),
            pl.BlockSpec((1, D), lambda i: (0, 0)),
        ],
        out_specs=pl.BlockSpec((bn, D), lambda i: (i, 0)),
        out_shape=jax.ShapeDtypeStruct((N, D), jnp.float32),
    )(x, w1, acc, sc, bias.reshape(1, D))


def kernel(x, edge_index, edge_attr, weight, bias):
    row = edge_index[0].reshape(E // B, B)
    col = edge_index[1].reshape(E // B, B)
    ea = edge_attr.reshape(E // B, B)
    w1 = weight[:D]
    w2 = weight[D:]
    w2h = jnp.stack([w2[:, :DH], w2[:, DH:]])
    y = _matmul_halves(x, w2h).reshape(NC * N, DH)
    acc, sc = _sc_edge_scatter(y, row, col, ea)
    return _finale(x, w1, acc, sc, bias)


# speedup vs baseline: 12.0173x; 1.8384x over previous
import functools

import jax
import jax.numpy as jnp
from jax import lax
from jax.experimental import pallas as pl
from jax.experimental.pallas import tpu as pltpu
from jax.experimental.pallas import tpu_sc as plsc

NC = 2
NS = 16
L = 16

N = 10000
E = 320000
D = 128
DH = D // NC

EPT = E // NS
B = 80
NB = EPT // B
NPT = 624
TAIL = N - NS * NPT
ZR = 104

_mesh = plsc.VectorSubcoreMesh(core_axis_name="c", subcore_axis_name="s")


@functools.partial(
    pl.kernel,
    out_type=(
        jax.ShapeDtypeStruct((NC, N, DH), jnp.float32),
        jax.ShapeDtypeStruct((NC, N, L), jnp.float32),
    ),
    mesh=_mesh,
    compiler_params=pltpu.CompilerParams(use_tc_tiling_on_sc=False),
    scratch_types=(
        pltpu.VMEM((NB, B), jnp.int32),
        pltpu.VMEM((NB, B), jnp.int32),
        pltpu.VMEM((NB, B), jnp.float32),
        pltpu.VMEM((2, B, DH), jnp.float32),
        pltpu.VMEM((B, L), jnp.float32),
        pltpu.VMEM((ZR, DH), jnp.float32),
        pltpu.VMEM((ZR, L), jnp.float32),
        pltpu.VMEM_SHARED((N, DH), jnp.float32),
        pltpu.VMEM_SHARED((N, L), jnp.float32),
        pltpu.SemaphoreType.DMA,
        pltpu.SemaphoreType.DMA,
        pltpu.SemaphoreType.DMA,
        pltpu.SemaphoreType.DMA,
        pltpu.SemaphoreType.DMA,
        pltpu.SemaphoreType.DMA,
    ),
)
def _sc_edge_scatter(y_hbm, row_hbm, col_hbm, ea_hbm, acc_out, sc_out,
                     row_v, col_v, ea_v, rows_v, scal_v, zrow_v, zsc_v,
                     acc_sh, sc_sh, g0, g1, s0, s1, ss, pre):
    cid = lax.axis_index("c")
    sid = lax.axis_index("s")
    gsem = (g0, g1)
    ssem = (s0, s1)

    cbase = pl.multiple_of(sid * NB, 2)
    d_row = pltpu.async_copy(row_hbm.at[pl.ds(cbase, NB)], row_v, pre)
    d_col = pltpu.async_copy(col_hbm.at[pl.ds(cbase, NB)], col_v, pre)
    d_ea = pltpu.async_copy(ea_hbm.at[pl.ds(cbase, NB)], ea_v, pre)

    def _zero_fill(i, c):
        zero = jnp.zeros((L,), jnp.float32)
        for j in range(DH // L):
            zrow_v[i, pl.ds(j * L, L)] = zero
        zsc_v[i, :] = zero
        return c

    lax.fori_loop(0, ZR, _zero_fill, 0)

    nbase = pl.multiple_of(sid * NPT, 8)
    for k in range(NPT // ZR):
        pltpu.sync_copy(zrow_v, acc_sh.at[pl.ds(nbase + k * ZR, ZR)])
        pltpu.sync_copy(zsc_v, sc_sh.at[pl.ds(nbase + k * ZR, ZR)])

    @pl.when(sid == NS - 1)
    def _zero_tail():
        pltpu.sync_copy(zrow_v.at[pl.ds(0, TAIL)], acc_sh.at[pl.ds(NS * NPT, TAIL)])
        pltpu.sync_copy(zsc_v.at[pl.ds(0, TAIL)], sc_sh.at[pl.ds(NS * NPT, TAIL)])

    d_row.wait()
    d_col.wait()
    d_ea.wait()

    def _rebase(g, c):
        gbase = pl.multiple_of(g * L, L)
        m = g // (B // L)
        o = (g % (B // L)) * L
        col_v[m, pl.ds(o, L)] = col_v[m, pl.ds(o, L)] + cid * N
        return c

    lax.fori_loop(0, NB * (B // L), _rebase, 0)
    plsc.subcore_barrier()

    pltpu.async_copy(y_hbm.at[col_v.at[0]], rows_v.at[0], g0)

    def _outer(m, c):
        for b in range(2):
            k = 2 * m + b
            nb = 1 - b

            @pl.when(k >= 1)
            def _drain_prev_scatter():
                pltpu.make_async_copy(
                    rows_v.at[nb], acc_sh.at[row_v.at[k - 1]], ssem[nb]).wait()

            @pl.when(k + 1 < NB)
            def _start_next_gather():
                pltpu.async_copy(y_hbm.at[col_v.at[k + 1]], rows_v.at[nb], gsem[nb])

            pltpu.make_async_copy(
                y_hbm.at[col_v.at[k]], rows_v.at[b], gsem[b]).wait()

            @plsc.parallel_loop(0, B // L)
            def _group(g):
                gbase = pl.multiple_of(g * L, L)
                ea16 = ea_v[k, pl.ds(gbase, L)]
                for l in range(L):
                    e = gbase + l
                    ea_bc = jnp.full((L,), ea16[l], jnp.float32)
                    for j in range(DH // L):
                        rows_v[b, e, pl.ds(j * L, L)] = (
                            rows_v[b, e, pl.ds(j * L, L)] * ea_bc)
            pltpu.async_copy(rows_v.at[b], acc_sh.at[row_v.at[k]], ssem[b],
                             add=True)

            @pl.when(k % NC == cid)
            def _scal():
                @pl.when(k >= 2)
                def _drain_prev():
                    pltpu.make_async_copy(
                        scal_v, sc_sh.at[row_v.at[k]], ss).wait()

                @plsc.parallel_loop(0, B // L)
                def _sgroup(g):
                    gbase = pl.multiple_of(g * L, L)
                    ea16 = ea_v[k, pl.ds(gbase, L)]
                    iota = lax.iota(jnp.int32, L)
                    for l in range(L):
                        ea_bc = jnp.full((L,), ea16[l], jnp.float32)
                        scal_v[gbase + l, :] = jnp.where(
                            iota == 0, ea_bc, jnp.where(iota == 1, 1.0, 0.0))

                pltpu.async_copy(scal_v, sc_sh.at[row_v.at[k]], ss, add=True)

        return c

    lax.fori_loop(0, NB // 2, _outer, 0)

    pltpu.make_async_copy(rows_v.at[1], acc_sh.at[row_v.at[NB - 1]], s1).wait()
    pltpu.make_async_copy(scal_v, sc_sh.at[row_v.at[NB - 1]], ss).wait()
    plsc.subcore_barrier()

    pltpu.sync_copy(acc_sh.at[pl.ds(nbase, NPT)], acc_out.at[cid, pl.ds(nbase, NPT)])
    pltpu.sync_copy(sc_sh.at[pl.ds(nbase, NPT)], sc_out.at[cid, pl.ds(nbase, NPT)])

    @pl.when(sid == NS - 1)
    def _write_tail():
        pltpu.sync_copy(acc_sh.at[pl.ds(NS * NPT, TAIL)],
                        acc_out.at[cid, pl.ds(NS * NPT, TAIL)])
        pltpu.sync_copy(sc_sh.at[pl.ds(NS * NPT, TAIL)],
                        sc_out.at[cid, pl.ds(NS * NPT, TAIL)])


def _mm_body(x_ref, w_ref, o_ref):
    o_ref[0] = jnp.dot(x_ref[...], w_ref[0], preferred_element_type=jnp.float32)


def _matmul_halves(x, w2h):
    g = 10
    bn = N // g
    return pl.pallas_call(
        _mm_body,
        grid=(NC, g),
        in_specs=[pl.BlockSpec((bn, D), lambda c, i: (i, 0)),
                  pl.BlockSpec((1, D, DH), lambda c, i: (c, 0, 0))],
        out_specs=pl.BlockSpec((1, bn, DH), lambda c, i: (c, i, 0)),
        out_shape=jax.ShapeDtypeStruct((NC, N, DH), jnp.float32),
    )(x, w2h)


def _fin_body(x_ref, w_ref, acc_ref, sc_ref, b_ref, o_ref):
    z1 = jnp.dot(x_ref[...], w_ref[...], preferred_element_type=jnp.float32)
    scat = jnp.concatenate([acc_ref[0], acc_ref[1]], axis=-1)
    srow = sc_ref[0] + sc_ref[1]
    sea = srow[:, 0:1]
    cnt = jnp.maximum(srow[:, 1:2], 1.0)
    o_ref[...] = (sea * z1 + scat) / cnt + b_ref[...]


def _finale(x, w1, acc, sc, bias):
    g = 10
    bn = N // g
    return pl.pallas_call(
        _fin_body,
        grid=(g,),
        in_specs=[
            pl.BlockSpec((bn, D), lambda i: (i, 0)),
            pl.BlockSpec((D, D), lambda i: (0, 0)),
            pl.BlockSpec((NC, bn, DH), lambda i: (0, i, 0)),
            pl.BlockSpec((NC, bn, L), lambda i: (0, i, 0)),
            pl.BlockSpec((1, D), lambda i: (0, 0)),
        ],
        out_specs=pl.BlockSpec((bn, D), lambda i: (i, 0)),
        out_shape=jax.ShapeDtypeStruct((N, D), jnp.float32),
    )(x, w1, acc, sc, bias.reshape(1, D))


def kernel(x, edge_index, edge_attr, weight, bias):
    row = edge_index[0].reshape(E // B, B)
    col = edge_index[1].reshape(E // B, B)
    ea = edge_attr.reshape(E // B, B)
    w1 = weight[:D]
    w2 = weight[D:]
    w2h = jnp.stack([w2[:, :DH], w2[:, DH:]])
    y = _matmul_halves(x, w2h).reshape(NC * N, DH)
    acc, sc = _sc_edge_scatter(y, row, col, ea)
    return _finale(x, w1, acc, sc, bias)
